# baseline (speedup 1.0000x reference)
import functools

import jax
import jax.numpy as jnp
import numpy as np
from jax.experimental import pallas as pl
from jax.experimental.pallas import tpu as pltpu
from jax.experimental.pallas import tpu_sc as plsc

EMBED_DIM = 2048
INNER_DIM = 64
PRUNE_RATIO = 0.25
_SQRT_HALF = float(np.sqrt(0.5).astype(np.float32))
_INV_C = float(np.float32(1.0 / 2048.0))

_ERFC_P = [2.326819970068386e-2, -1.387039388740657e-1, 3.687424674597105e-1,
           -5.824733027278666e-1, 6.210004621745983e-1, -4.944515323274145e-1,
           3.404879937665872e-1, -2.741127028184656e-1, 5.638259427386472e-1]
_ERFC_R = [-1.047766399936249e+1, 1.297719955372516e+1, -7.495518717768503e+0,
           2.921019019210786e+0, -1.015265279202700e+0, 4.218463358204948e-1,
           -2.820767439740514e-1, 5.641895067754075e-1]
_ERF_T = [7.853861353153693e-5, -8.010193625184903e-4, 5.188327685732524e-3,
          -2.685381193529856e-2, 1.128358514861418e-1, -3.761262582423300e-1,
          1.128379165726710e+0]


def _poly(y, cs):
    p = jnp.full_like(y, np.float32(cs[0]))
    for c in cs[1:]:
        p = p * y + np.float32(c)
    return p


def _gelu_exact(x):
    a = x * np.float32(-_SQRT_HALF)
    a2 = a * a
    neg_a2 = -a2
    abs_a = jnp.abs(a)
    z = jnp.exp(neg_a2)
    y = 1.0 / a2
    q = 1.0 / abs_a
    p = jnp.where(abs_a < 2.0, _poly(y, _ERFC_P), _poly(y, _ERFC_R))
    yv = (z * q) * p
    yv = jnp.where(neg_a2 < np.float32(-88.72284), 0.0, yv)
    tail = jnp.where(a < 0.0, 2.0 - yv, yv)
    erf_small = a * _poly(a2, _ERF_T)
    erfc = jnp.where(abs_a < 1.0, 1.0 - erf_small, tail)
    return (0.5 * x) * erfc


def _row_sum(t):
    n = t.shape[1]
    acc = t[:, 0:128]
    for k in range(1, n // 128):
        acc = acc + t[:, k * 128:(k + 1) * 128]
    b = acc[:, 0:8]
    for s in range(1, 16):
        b = b + acc[:, s * 8:(s + 1) * 8]
    t1 = b[:, 0:4] + b[:, 4:8]
    t2 = t1[:, 0:2] + t1[:, 2:4]
    return t2[:, 0:1] + t2[:, 1:2]

_SCORE_BLK = 256
_RANK_BLK = 512
_PERM_BLK = 512
_GATHER_WIN = 16


def _scores_body(x_ref, lnw_ref, lnb_ref, w1_ref, b1_ref, w2_ref, b2_ref,
                 out_ref):
    xb = x_ref[0]
    mean = _row_sum(xb) * np.float32(_INV_C)
    d = xb - mean
    varsum = _row_sum(d * d)
    vp = np.float32(1e-5) + varsum * np.float32(_INV_C)
    std = vp * jax.lax.rsqrt(vp)
    h = ((d * (1.0 / std)) * lnw_ref[...]) + lnb_ref[...]
    h1 = jnp.dot(h, w1_ref[...], preferred_element_type=jnp.float32)
    h1 = h1 + b1_ref[...]
    g = _gelu_exact(h1)
    h2 = jnp.dot(g, w2_ref[...], preferred_element_type=jnp.float32)
    h2 = h2 + b2_ref[...]
    out_ref[...] = h2.reshape(1, h2.shape[0], 1)


def _rank_body(scol_ref, srow_ref, out_ref):
    blk_i = pl.program_id(1)
    s_col = scol_ref[0]
    s_row = srow_ref[0]
    n = srow_ref.shape[2]
    m = jnp.max(s_row, axis=1, keepdims=True)
    e_row = jnp.exp(s_row - m)
    r = 1.0 / _row_sum(e_row)
    k_row = e_row * r
    k_col = jnp.exp(s_col - m) * r
    i_idx = (jax.lax.broadcasted_iota(jnp.int32, (_RANK_BLK, n), 0)
             + blk_i * _RANK_BLK)
    j_idx = jax.lax.broadcasted_iota(jnp.int32, (_RANK_BLK, n), 1)
    gt = (k_row > k_col) | ((k_row == k_col) & (j_idx < i_idx))
    rank = jnp.sum(gt.astype(jnp.int32), axis=1, keepdims=True)
    out_ref[...] = rank.reshape(1, _RANK_BLK, 1)


def _perm_body(rrow_ref, out_ref):
    b = pl.program_id(0)
    blk_p = pl.program_id(1)
    r_row = rrow_ref[0]
    n = rrow_ref.shape[2]
    p_idx = (jax.lax.broadcasted_iota(jnp.int32, (_PERM_BLK, n), 0)
             + blk_p * _PERM_BLK)
    j_idx = jax.lax.broadcasted_iota(jnp.int32, (_PERM_BLK, n), 1)
    eq = (r_row == p_idx).astype(jnp.int32)
    perm = jnp.sum(eq * j_idx, axis=1, keepdims=True) + b * n
    out_ref[...] = perm.reshape(1, _PERM_BLK, 1)


def _compute_scores(x, ln_w, ln_b, w1, b1, w2, b2):
    B, N, C = x.shape
    grid = (B, N // _SCORE_BLK)
    return pl.pallas_call(
        _scores_body,
        grid=grid,
        in_specs=[
            pl.BlockSpec((1, _SCORE_BLK, C), lambda b, i: (b, i, 0)),
            pl.BlockSpec((1, C), lambda b, i: (0, 0)),
            pl.BlockSpec((1, C), lambda b, i: (0, 0)),
            pl.BlockSpec((C, INNER_DIM), lambda b, i: (0, 0)),
            pl.BlockSpec((1, INNER_DIM), lambda b, i: (0, 0)),
            pl.BlockSpec((INNER_DIM, 1), lambda b, i: (0, 0)),
            pl.BlockSpec((1, 1), lambda b, i: (0, 0)),
        ],
        out_specs=pl.BlockSpec((1, _SCORE_BLK, 1), lambda b, i: (b, i, 0)),
        out_shape=jax.ShapeDtypeStruct((B, N, 1), jnp.float32),
    )(x, ln_w.reshape(1, C), ln_b.reshape(1, C), w1, b1.reshape(1, INNER_DIM),
      w2, b2.reshape(1, 1))


def _compute_ranks(scores_col, scores_row):
    B, N, _ = scores_col.shape
    return pl.pallas_call(
        _rank_body,
        grid=(B, N // _RANK_BLK),
        in_specs=[
            pl.BlockSpec((1, _RANK_BLK, 1), lambda b, i: (b, i, 0)),
            pl.BlockSpec((1, 1, N), lambda b, i: (b, 0, 0)),
        ],
        out_specs=pl.BlockSpec((1, _RANK_BLK, 1), lambda b, i: (b, i, 0)),
        out_shape=jax.ShapeDtypeStruct((B, N, 1), jnp.int32),
    )(scores_col, scores_row)


def _compute_perm(rank_row, n_prune):
    B, _, N = rank_row.shape
    return pl.pallas_call(
        _perm_body,
        grid=(B, n_prune // _PERM_BLK),
        in_specs=[pl.BlockSpec((1, 1, N), lambda b, p: (b, 0, 0))],
        out_specs=pl.BlockSpec((1, _PERM_BLK, 1), lambda b, p: (b, p, 0)),
        out_shape=jax.ShapeDtypeStruct((B, n_prune, 1), jnp.int32),
    )(rank_row)


def _sc_gather(x2d, flat_idx):
    n_rows, C = flat_idx.shape[0], x2d.shape[1]
    mesh = plsc.VectorSubcoreMesh(core_axis_name="core",
                                  subcore_axis_name="subcore")
    n_w = 32
    per_w = n_rows // n_w
    ch = 32

    @functools.partial(
        pl.kernel,
        out_type=jax.ShapeDtypeStruct((n_rows, C), jnp.float32),
        mesh=mesh,
        scratch_types=[
            pltpu.VMEM((per_w,), jnp.int32),
            pltpu.VMEM((ch, C), jnp.float32),
            pltpu.SemaphoreType.DMA,
        ],
    )
    def gather_kernel(x_hbm, i_hbm, o_hbm, idx_v, rows_v, sem):
        wid = jax.lax.axis_index("subcore") * 2 + jax.lax.axis_index("core")
        base = wid * per_w
        pltpu.sync_copy(i_hbm.at[pl.ds(base, per_w)], idx_v)
        for c in range(per_w // ch):
            pltpu.async_copy(
                x_hbm.at[idx_v.at[pl.ds(c * ch, ch)]], rows_v, sem).wait()
            pltpu.sync_copy(rows_v, o_hbm.at[pl.ds(base + c * ch, ch)])

    return gather_kernel(x2d, flat_idx)


def kernel(x, ln_w, ln_b, w1, b1, w2, b2):
    B, N, C = x.shape
    n_prune = int(N * PRUNE_RATIO)
    scores_col = _compute_scores(x, ln_w, ln_b, w1, b1, w2, b2)
    scores_row = scores_col.reshape(B, 1, N)
    rank_col = _compute_ranks(scores_col, scores_row)
    rank_row = rank_col.reshape(B, 1, N)
    perm_col = _compute_perm(rank_row, n_prune)
    flat_idx = perm_col.reshape(B * n_prune)
    out2d = _sc_gather(x.reshape(B * N, C), flat_idx)
    return out2d.reshape(B, n_prune, C)
